# baseline (speedup 1.0000x reference)
import functools
import math

import jax
import jax.numpy as jnp
from jax import lax
from jax.experimental import pallas as pl
from jax.experimental.pallas import tpu as pltpu
from jax.experimental.pallas import tpu_sc as plsc

B = 16384
D = 64
V = 1000000
H = 1024
BETA = 0.95

NC = 2
NS = 16
NW = NC * NS
L = 16

RPT = V // NW
BPT = B // NW
NCH = 10
CR = RPT // NCH

_mesh = plsc.VectorSubcoreMesh(core_axis_name="c", subcore_axis_name="s")
_sc_params = pltpu.CompilerParams(use_tc_tiling_on_sc=False,
                                 needs_layout_passes=False)


@functools.partial(
    pl.kernel,
    out_type=(jax.ShapeDtypeStruct((B, D), jnp.float32),
              jax.ShapeDtypeStruct((B, D), jnp.float32)),
    mesh=_mesh,
    compiler_params=_sc_params,
    scratch_types=[
        pltpu.VMEM((BPT,), jnp.int32),
        pltpu.VMEM((BPT, D), jnp.float32),
        pltpu.VMEM((BPT, D), jnp.float32),
        pltpu.SemaphoreType.DMA,
        pltpu.SemaphoreType.DMA,
    ],
)
def _sc_gather(ind_hbm, aux_hbm, proto_hbm, v_out, z_out, idx_v, va, vb, s1, s2):
    wid = lax.axis_index("s") * NC + lax.axis_index("c")
    base = wid * BPT
    pltpu.sync_copy(ind_hbm.at[pl.ds(base, BPT)], idx_v)
    ca = pltpu.async_copy(aux_hbm.at[idx_v], va, s1)
    cb = pltpu.async_copy(proto_hbm.at[idx_v], vb, s2)
    ca.wait()
    cb.wait()
    pltpu.sync_copy(va, v_out.at[pl.ds(base, BPT)])
    pltpu.sync_copy(vb, z_out.at[pl.ds(base, BPT)])


def _vrsqrt(s):
    i = plsc.bitcast(s, jnp.int32)
    y = plsc.bitcast(jnp.int32(0x5F3759DF) - (i >> 1), jnp.float32)
    for _ in range(4):
        y = y * (1.5 - 0.5 * s * y * y)
    return y


def _update_rows16(rows_ref):
    for r in range(L):
        regs = [rows_ref[r, pl.ds(k * L, L)] for k in range(D // L)]
        t = regs[0] * regs[0]
        for q in regs[1:]:
            t = t + q * q
        s = jnp.maximum(jnp.sum(t), 1e-24)
        sv = jnp.full((L,), s, dtype=jnp.float32)
        scale = BETA + (1.0 - BETA) * _vrsqrt(sv)
        for k in range(D // L):
            rows_ref[r, pl.ds(k * L, L)] = regs[k] * scale


@functools.partial(
    pl.kernel,
    out_type=jax.ShapeDtypeStruct((V, D), jnp.float32),
    mesh=_mesh,
    compiler_params=_sc_params,
    scratch_types=[
        pltpu.VMEM((B,), jnp.int32),
        pltpu.VMEM((B + L,), jnp.int32),
        pltpu.VMEM((L,), jnp.int32),
        pltpu.VMEM((L, D), jnp.float32),
        pltpu.SemaphoreType.DMA,
        pltpu.SemaphoreType.DMA,
        pltpu.SemaphoreType.DMA,
    ],
)
def _sc_update(ind_hbm, proto_hbm, out_hbm, ind_v, sel_v, idx16, rows16,
               csem, gsem, ssem):
    wid = lax.axis_index("s") * NC + lax.axis_index("c")
    lo = wid * RPT
    hi = lo + RPT

    copies = [
        pltpu.async_copy(proto_hbm.at[pl.ds(lo + c * CR, CR)],
                         out_hbm.at[pl.ds(lo + c * CR, CR)], csem)
        for c in range(NCH)
    ]

    pltpu.sync_copy(ind_hbm, ind_v)

    def scan_body(i, cnt):
        iv = ind_v[pl.ds(i * L, L)]
        m = (iv >= lo) & (iv < hi)
        pos = plsc.cumsum(m.astype(jnp.int32))
        plsc.store_scatter(sel_v, [cnt + pos - 1], iv, mask=m)
        return cnt + jnp.sum(m.astype(jnp.int32))

    cnt = lax.fori_loop(0, B // L, scan_body, jnp.int32(0))

    for c in copies:
        c.wait()

    def group_body(j, carry):
        idx16[pl.ds(0, L)] = sel_v[pl.ds(j * L, L)]
        pltpu.async_copy(proto_hbm.at[idx16], rows16, gsem).wait()
        _update_rows16(rows16)
        pltpu.async_copy(rows16, out_hbm.at[idx16], ssem).wait()
        return carry

    lax.fori_loop(0, cnt // L, group_body, 0)

    def rem_body(k, carry):
        a = sel_v[pl.ds((cnt // L) * L + k, L)][0]
        pltpu.sync_copy(proto_hbm.at[pl.ds(a, 1)], rows16.at[pl.ds(0, 1)])
        regs = [rows16[0, pl.ds(q * L, L)] for q in range(D // L)]
        t = regs[0] * regs[0]
        for q in regs[1:]:
            t = t + q * q
        s = jnp.maximum(jnp.sum(t), 1e-24)
        sv = jnp.full((L,), s, dtype=jnp.float32)
        scale = BETA + (1.0 - BETA) * _vrsqrt(sv)
        for q in range(D // L):
            rows16[0, pl.ds(q * L, L)] = regs[q] * scale
        pltpu.sync_copy(rows16.at[pl.ds(0, 1)], out_hbm.at[pl.ds(a, 1)])
        return carry

    lax.fori_loop(0, cnt - (cnt // L) * L, rem_body, 0)


BB = 512
_INV_SQRT_D = 1.0 / math.sqrt(D)
_INV_SQRT_2 = 1.0 / math.sqrt(2.0)


def _ln(x, g, b, eps=1e-5):
    mu = jnp.mean(x, axis=-1, keepdims=True)
    xc = x - mu
    var = jnp.mean(xc * xc, axis=-1, keepdims=True)
    return xc * lax.rsqrt(var + eps) * g + b


def _tc_body(x_ref, v_ref, z_ref, wq, bq, wk, bk, wv, bv, wo, bo,
             w1, b1, w2, b2, g1, be1, g2, be2, out_ref):
    xb = x_ref[...]
    vb = v_ref[...]
    zb = z_ref[...]
    S = jnp.concatenate([xb, vb, zb], axis=0)
    f32 = jnp.float32
    Q = jnp.dot(S, wq[...], preferred_element_type=f32) + bq[...]
    K = jnp.dot(S, wk[...], preferred_element_type=f32) + bk[...]
    Vv = jnp.dot(S, wv[...], preferred_element_type=f32) + bv[...]
    q = [Q[i * BB:(i + 1) * BB] for i in range(3)]
    k = [K[i * BB:(i + 1) * BB] for i in range(3)]
    v = [Vv[i * BB:(i + 1) * BB] for i in range(3)]
    ao = []
    for i in range(3):
        sc = [jnp.sum(q[i] * k[j], axis=-1, keepdims=True) * _INV_SQRT_D
              for j in range(3)]
        m = jnp.maximum(jnp.maximum(sc[0], sc[1]), sc[2])
        e = [jnp.exp(s - m) for s in sc]
        den = e[0] + e[1] + e[2]
        ao.append((e[0] * v[0] + e[1] * v[1] + e[2] * v[2]) / den)
    AO = jnp.concatenate(ao, axis=0)
    P = jnp.dot(AO, wo[...], preferred_element_type=f32) + bo[...]
    H1 = _ln(S + P, g1[...], be1[...])
    U = jnp.dot(H1, w1[...], preferred_element_type=f32) + b1[...]
    U = 0.5 * U * (1.0 + lax.erf(U * _INV_SQRT_2))
    FF = jnp.dot(U, w2[...], preferred_element_type=f32) + b2[...]
    H2 = _ln(H1 + FF, g2[...], be2[...])
    out_ref[...] = (H2[0:BB] + H2[BB:2 * BB] + H2[2 * BB:3 * BB]) * (1.0 / 3.0)


def _tc_transformer(x, v, z, wq, bq, wk, bk, wv, bv, wo, bo,
                    w1, b1, w2, b2, g1, be1, g2, be2):
    bspec = pl.BlockSpec((BB, D), lambda i: (i, 0))
    full = lambda r, c: pl.BlockSpec((r, c), lambda i: (0, 0))
    return pl.pallas_call(
        _tc_body,
        grid=(B // BB,),
        in_specs=[
            bspec, bspec, bspec,
            full(D, D), full(1, D), full(D, D), full(1, D),
            full(D, D), full(1, D), full(D, D), full(1, D),
            full(D, H), full(1, H), full(H, D), full(1, D),
            full(1, D), full(1, D), full(1, D), full(1, D),
        ],
        out_specs=bspec,
        out_shape=jax.ShapeDtypeStruct((B, D), jnp.float32),
    )(x, v, z, wq, bq, wk, bk, wv, bv, wo, bo, w1, b1, w2, b2,
      g1, be1, g2, be2)


def kernel(x, ind, aux_table, proto_table, Wq, bq, Wk, bk, Wv, bv, Wo, bo,
           W1, b1, W2, b2, ln1_g, ln1_b, ln2_g, ln2_b):
    ind32 = ind.astype(jnp.int32)
    v, z = _sc_gather(ind32, aux_table, proto_table)
    enriched = _tc_transformer(
        x, v, z, Wq, bq.reshape(1, D), Wk, bk.reshape(1, D),
        Wv, bv.reshape(1, D), Wo, bo.reshape(1, D),
        W1, b1.reshape(1, H), W2, b2.reshape(1, D),
        ln1_g.reshape(1, D), ln1_b.reshape(1, D),
        ln2_g.reshape(1, D), ln2_b.reshape(1, D))
    new_proto = _sc_update(ind32, proto_table)
    return enriched, new_proto


# speedup vs baseline: 5.2416x; 5.2416x over previous
import functools
import math

import jax
import jax.numpy as jnp
from jax import lax
from jax.experimental import pallas as pl
from jax.experimental.pallas import tpu as pltpu
from jax.experimental.pallas import tpu_sc as plsc

B = 16384
D = 64
V = 1000000
H = 1024
BETA = 0.95

NC = 2
NS = 16
NW = NC * NS
L = 16

RPT = V // NW
BPT = B // NW
NCH = 10
CR = RPT // NCH

_mesh = plsc.VectorSubcoreMesh(core_axis_name="c", subcore_axis_name="s")
_sc_params = pltpu.CompilerParams(use_tc_tiling_on_sc=False,
                                 needs_layout_passes=False)


@functools.partial(
    pl.kernel,
    out_type=(jax.ShapeDtypeStruct((B, D), jnp.float32),
              jax.ShapeDtypeStruct((B, D), jnp.float32)),
    mesh=_mesh,
    compiler_params=_sc_params,
    scratch_types=[
        pltpu.VMEM((BPT,), jnp.int32),
        pltpu.VMEM((BPT, D), jnp.float32),
        pltpu.VMEM((BPT, D), jnp.float32),
        pltpu.SemaphoreType.DMA,
        pltpu.SemaphoreType.DMA,
    ],
)
def _sc_gather(ind_hbm, aux_hbm, proto_hbm, v_out, z_out, idx_v, va, vb, s1, s2):
    wid = lax.axis_index("s") * NC + lax.axis_index("c")
    base = wid * BPT
    pltpu.sync_copy(ind_hbm.at[pl.ds(base, BPT)], idx_v)
    ca = pltpu.async_copy(aux_hbm.at[idx_v], va, s1)
    cb = pltpu.async_copy(proto_hbm.at[idx_v], vb, s2)
    ca.wait()
    cb.wait()
    pltpu.sync_copy(va, v_out.at[pl.ds(base, BPT)])
    pltpu.sync_copy(vb, z_out.at[pl.ds(base, BPT)])


def _vrsqrt(s):
    i = plsc.bitcast(s, jnp.int32)
    y = plsc.bitcast(jnp.int32(0x5F3759DF) - (i >> 1), jnp.float32)
    for _ in range(4):
        y = y * (1.5 - 0.5 * s * y * y)
    return y


def _update_rows16(rows_ref):
    for r in range(L):
        regs = [rows_ref[r, pl.ds(k * L, L)] for k in range(D // L)]
        t = regs[0] * regs[0]
        for q in regs[1:]:
            t = t + q * q
        s = jnp.maximum(jnp.sum(t), 1e-24)
        sv = jnp.full((L,), s, dtype=jnp.float32)
        scale = BETA + (1.0 - BETA) * _vrsqrt(sv)
        for k in range(D // L):
            rows_ref[r, pl.ds(k * L, L)] = regs[k] * scale


def _upd_body(ind2_hbm, z_hbm, proto_hbm, out_hbm, idx2d, rows_v, ssem):
    wid = lax.axis_index("s") * NC + lax.axis_index("c")
    base = wid * BPT
    pltpu.sync_copy(ind2_hbm.at[pl.ds(wid * (BPT // 128), BPT // 128)], idx2d)
    pltpu.sync_copy(z_hbm.at[pl.ds(base, BPT)], rows_v)

    def g16(g, carry):
        for r in range(L):
            row = g * L + r
            regs = [rows_v[row, pl.ds(k * L, L)] for k in range(D // L)]
            t = regs[0] * regs[0]
            for q in regs[1:]:
                t = t + q * q
            s = jnp.maximum(jnp.sum(t), 1e-24)
            sv = jnp.full((L,), s, dtype=jnp.float32)
            scale = BETA + (1.0 - BETA) * _vrsqrt(sv)
            for k in range(D // L):
                rows_v[row, pl.ds(k * L, L)] = regs[k] * scale
        return carry

    lax.fori_loop(0, BPT // L, g16, 0)
    scat = [
        pltpu.async_copy(rows_v.at[pl.ds(j * 128, 128)],
                         out_hbm.at[idx2d.at[j]], ssem)
        for j in range(BPT // 128)
    ]
    for s_ in scat:
        s_.wait()


from jax._src.pallas import mpmd as _mpmd

_sc_update = _mpmd._mpmd_map(
    [(_mesh, _upd_body)],
    (jax.ShapeDtypeStruct((V, D), jnp.float32),),
    input_output_aliases={2: 0},
    compiler_params=_sc_params,
    scratch_types=[
        pltpu.VMEM((BPT // 128, 128), jnp.int32),
        pltpu.VMEM((BPT, D), jnp.float32),
        pltpu.SemaphoreType.DMA,
    ],
)


BB = 512
_INV_SQRT_D = 1.0 / math.sqrt(D)
_INV_SQRT_2 = 1.0 / math.sqrt(2.0)


def _ln(x, g, b, eps=1e-5):
    mu = jnp.mean(x, axis=-1, keepdims=True)
    xc = x - mu
    var = jnp.mean(xc * xc, axis=-1, keepdims=True)
    return xc * lax.rsqrt(var + eps) * g + b


def _tc_body(x_ref, v_ref, z_ref, wq, bq, wk, bk, wv, bv, wo, bo,
             w1, b1, w2, b2, g1, be1, g2, be2, out_ref):
    xb = x_ref[...]
    vb = v_ref[...]
    zb = z_ref[...]
    S = jnp.concatenate([xb, vb, zb], axis=0)
    f32 = jnp.float32
    Q = jnp.dot(S, wq[...], preferred_element_type=f32) + bq[...]
    K = jnp.dot(S, wk[...], preferred_element_type=f32) + bk[...]
    Vv = jnp.dot(S, wv[...], preferred_element_type=f32) + bv[...]
    q = [Q[i * BB:(i + 1) * BB] for i in range(3)]
    k = [K[i * BB:(i + 1) * BB] for i in range(3)]
    v = [Vv[i * BB:(i + 1) * BB] for i in range(3)]
    ao = []
    for i in range(3):
        sc = [jnp.sum(q[i] * k[j], axis=-1, keepdims=True) * _INV_SQRT_D
              for j in range(3)]
        m = jnp.maximum(jnp.maximum(sc[0], sc[1]), sc[2])
        e = [jnp.exp(s - m) for s in sc]
        den = e[0] + e[1] + e[2]
        ao.append((e[0] * v[0] + e[1] * v[1] + e[2] * v[2]) / den)
    AO = jnp.concatenate(ao, axis=0)
    P = jnp.dot(AO, wo[...], preferred_element_type=f32) + bo[...]
    H1 = _ln(S + P, g1[...], be1[...])
    U = jnp.dot(H1, w1[...], preferred_element_type=f32) + b1[...]
    U = 0.5 * U * (1.0 + lax.erf(U * _INV_SQRT_2))
    FF = jnp.dot(U, w2[...], preferred_element_type=f32) + b2[...]
    H2 = _ln(H1 + FF, g2[...], be2[...])
    out_ref[...] = (H2[0:BB] + H2[BB:2 * BB] + H2[2 * BB:3 * BB]) * (1.0 / 3.0)


def _tc_transformer(x, v, z, wq, bq, wk, bk, wv, bv, wo, bo,
                    w1, b1, w2, b2, g1, be1, g2, be2):
    bspec = pl.BlockSpec((BB, D), lambda i: (i, 0))
    full = lambda r, c: pl.BlockSpec((r, c), lambda i: (0, 0))
    return pl.pallas_call(
        _tc_body,
        grid=(B // BB,),
        in_specs=[
            bspec, bspec, bspec,
            full(D, D), full(1, D), full(D, D), full(1, D),
            full(D, D), full(1, D), full(D, D), full(1, D),
            full(D, H), full(1, H), full(H, D), full(1, D),
            full(1, D), full(1, D), full(1, D), full(1, D),
        ],
        out_specs=bspec,
        out_shape=jax.ShapeDtypeStruct((B, D), jnp.float32),
    )(x, v, z, wq, bq, wk, bk, wv, bv, wo, bo, w1, b1, w2, b2,
      g1, be1, g2, be2)


def kernel(x, ind, aux_table, proto_table, Wq, bq, Wk, bk, Wv, bv, Wo, bo,
           W1, b1, W2, b2, ln1_g, ln1_b, ln2_g, ln2_b):
    ind32 = ind.astype(jnp.int32)
    v, z = _sc_gather(ind32, aux_table, proto_table)
    enriched = _tc_transformer(
        x, v, z, Wq, bq.reshape(1, D), Wk, bk.reshape(1, D),
        Wv, bv.reshape(1, D), Wo, bo.reshape(1, D),
        W1, b1.reshape(1, H), W2, b2.reshape(1, D),
        ln1_g.reshape(1, D), ln1_b.reshape(1, D),
        ln2_g.reshape(1, D), ln2_b.reshape(1, D))
    ind2 = ind32.reshape(B // 128, 128)
    (new_proto,) = _sc_update(ind2, z, proto_table)
    return enriched, new_proto
